# baseline (speedup 1.0000x reference)
import functools

import jax
import jax.numpy as jnp
from jax import lax
from jax.experimental import pallas as pl
from jax.experimental.pallas import tpu as pltpu
from jax.experimental.pallas import tpu_sc as plsc

N = 10000
E = 160000
D = 256
H = 512
DH = D // 2

NC = 2
NS = 16
CHUNK = 128
EPT = 10240
NCHUNK = EPT // CHUNK
GRP = 16
NGRP = NCHUNK // GRP
NROW = N + NS
ROW_STEP = 624
ROW_LEN = 640

_sc_mesh = plsc.VectorSubcoreMesh(core_axis_name="c", subcore_axis_name="s")


@functools.partial(
    pl.kernel,
    out_type=jax.ShapeDtypeStruct((NC, N, DH), jnp.float32),
    mesh=_sc_mesh,
    scratch_types=[
        pltpu.VMEM((GRP, CHUNK), jnp.int32),
        pltpu.VMEM((GRP, CHUNK), jnp.int32),
        pltpu.VMEM((GRP, CHUNK), jnp.int32),
        pltpu.VMEM((GRP, CHUNK), jnp.int32),
        pltpu.VMEM((CHUNK, DH), jnp.float32),
        pltpu.VMEM((CHUNK, DH), jnp.float32),
        pltpu.VMEM_SHARED((NROW, DH), jnp.float32),
        pltpu.SemaphoreType.DMA,
        pltpu.SemaphoreType.DMA,
        pltpu.SemaphoreType.DMA,
        pltpu.SemaphoreType.DMA,
    ],
)
def _sc_aggregate(src_hbm, dst_hbm, h2_hbm, zeros_hbm, out_hbm,
                  src_a, dst_a, src_b, dst_b, rows_v0, rows_v1, agg_sh,
                  sem0, sem1, sem_ia, sem_ib):
    c = lax.axis_index("c")
    s = lax.axis_index("s")
    row0 = s * ROW_STEP

    def stage_group(t, sv, dv, sem):
        g = lax.min(t, NGRP - 1)
        pltpu.async_copy(src_hbm.at[c, s, pl.ds(g * GRP, GRP)], sv, sem)
        pltpu.async_copy(dst_hbm.at[s, pl.ds(g * GRP, GRP)], dv, sem)

    def wait_group(sv, dv, sem):
        pltpu.make_async_copy(src_hbm.at[c, s, pl.ds(0, GRP)], sv, sem).wait()
        pltpu.make_async_copy(dst_hbm.at[s, pl.ds(0, GRP)], dv, sem).wait()

    def run_block(sv, dv):
        pltpu.async_copy(h2_hbm.at[sv.at[0]], rows_v0, sem0)

        def u_body(u, carry):
            j0 = 2 * u
            j1 = j0 + 1
            pltpu.async_copy(h2_hbm.at[sv.at[j1]], rows_v1, sem1)
            pltpu.make_async_copy(h2_hbm.at[sv.at[j0]], rows_v0, sem0).wait()
            pltpu.sync_copy(rows_v0, agg_sh.at[dv.at[j0]], add=True)

            @pl.when(u < GRP // 2 - 1)
            def _():
                pltpu.async_copy(h2_hbm.at[sv.at[j0 + 2]], rows_v0, sem0)

            pltpu.make_async_copy(h2_hbm.at[sv.at[j1]], rows_v1, sem1).wait()
            pltpu.sync_copy(rows_v1, agg_sh.at[dv.at[j1]], add=True)
            return carry

        lax.fori_loop(0, GRP // 2, u_body, 0)

    stage_group(0, src_a, dst_a, sem_ia)
    pltpu.sync_copy(zeros_hbm, agg_sh.at[pl.ds(row0, ROW_LEN)])
    plsc.subcore_barrier()

    def group_body(t, carry):
        @pl.when(t % 2 == 0)
        def _():
            stage_group(t + 1, src_b, dst_b, sem_ib)
            wait_group(src_a, dst_a, sem_ia)
            run_block(src_a, dst_a)

        @pl.when(t % 2 == 1)
        def _():
            stage_group(t + 1, src_a, dst_a, sem_ia)
            wait_group(src_b, dst_b, sem_ib)
            run_block(src_b, dst_b)

        return carry

    lax.fori_loop(0, NGRP, group_body, 0)
    wait_group(src_b, dst_b, sem_ib)
    plsc.subcore_barrier()

    pltpu.sync_copy(agg_sh.at[pl.ds(row0, ROW_LEN)],
                    out_hbm.at[c, pl.ds(row0, ROW_LEN)])


_ROW_BLK = 1000


def _tc_matmul_body(a0_ref, a1_ref, wl_ref, wh_ref, b_ref, o_ref):
    acc = lax.dot_general(a0_ref[...], wl_ref[...],
                          (((1,), (1,)), ((), ())),
                          preferred_element_type=jnp.float32)
    acc += lax.dot_general(a1_ref[...], wh_ref[...],
                           (((1,), (1,)), ((), ())),
                           preferred_element_type=jnp.float32)
    o_ref[...] = acc + b_ref[...]


@jax.jit
def kernel(edge_index, h, W, b):
    src = edge_index[0]
    dst = edge_index[1]
    src_t = jnp.pad(src.reshape(NS, E // NS), ((0, 0), (0, EPT - E // NS)))
    trash = jnp.broadcast_to((N + jnp.arange(NS, dtype=jnp.int32))[:, None],
                             (NS, EPT - E // NS))
    dst_t = jnp.concatenate([dst.reshape(NS, E // NS), trash], axis=1)
    src2 = jnp.stack([2 * src_t, 2 * src_t + 1]).reshape(NC, NS, NCHUNK, CHUNK)
    dst_t = dst_t.reshape(NS, NCHUNK, CHUNK)
    h2 = h.reshape(2 * N, DH)
    zeros = jnp.zeros((ROW_LEN, DH), jnp.float32)

    agg2 = _sc_aggregate(src2, dst_t, h2, zeros)

    out = pl.pallas_call(
        _tc_matmul_body,
        grid=(N // _ROW_BLK,),
        in_specs=[
            pl.BlockSpec((_ROW_BLK, DH), lambda i: (i, 0)),
            pl.BlockSpec((_ROW_BLK, DH), lambda i: (i, 0)),
            pl.BlockSpec((H, DH), lambda i: (0, 0)),
            pl.BlockSpec((H, DH), lambda i: (0, 0)),
            pl.BlockSpec((1, H), lambda i: (0, 0)),
        ],
        out_specs=pl.BlockSpec((_ROW_BLK, H), lambda i: (i, 0)),
        out_shape=jax.ShapeDtypeStruct((N, H), jnp.float32),
    )(agg2[0], agg2[1], W[:, :DH], W[:, DH:], b.reshape(1, H))
    return out


# speedup vs baseline: 1.3201x; 1.3201x over previous
import functools

import jax
import jax.numpy as jnp
from jax import lax
from jax.experimental import pallas as pl
from jax.experimental.pallas import tpu as pltpu
from jax.experimental.pallas import tpu_sc as plsc

N = 10000
E = 160000
D = 256
H = 512
DH = D // 2

NC = 2
NS = 16
CHUNK = 125
EPT = E // NS
NCHUNK = EPT // CHUNK
NROW = N
ROW_STEP = 624
ROW_LEN = 640

_sc_mesh = plsc.VectorSubcoreMesh(core_axis_name="c", subcore_axis_name="s")


@functools.partial(
    pl.kernel,
    out_type=jax.ShapeDtypeStruct((NC, N, DH), jnp.float32),
    mesh=_sc_mesh,
    scratch_types=[
        pltpu.VMEM((NCHUNK, CHUNK), jnp.int32),
        pltpu.VMEM((NCHUNK, CHUNK), jnp.int32),
        pltpu.VMEM((CHUNK, DH), jnp.float32),
        pltpu.VMEM_SHARED((NROW, DH), jnp.float32),
        pltpu.SemaphoreType.DMA,
    ],
)
def _sc_aggregate(src_hbm, dst_hbm, h2_hbm, zeros_hbm, out_hbm,
                  src_v, dst_v, rows_v, agg_sh, sem):
    c = lax.axis_index("c")
    s = lax.axis_index("s")
    row0 = s * ROW_STEP

    pltpu.sync_copy(src_hbm.at[c, s], src_v)
    pltpu.sync_copy(dst_hbm.at[s], dst_v)
    pltpu.sync_copy(zeros_hbm, agg_sh.at[pl.ds(row0, ROW_LEN)])
    plsc.subcore_barrier()

    def chunk_body(j, carry):
        pltpu.async_copy(h2_hbm.at[src_v.at[j]], rows_v, sem).wait()
        pltpu.sync_copy(rows_v, agg_sh.at[dst_v.at[j]], add=True)
        return carry

    lax.fori_loop(0, NCHUNK, chunk_body, 0)
    plsc.subcore_barrier()

    pltpu.sync_copy(agg_sh.at[pl.ds(row0, ROW_LEN)],
                    out_hbm.at[c, pl.ds(row0, ROW_LEN)])


_ROW_BLK = 1000


def _tc_matmul_body(a0_ref, a1_ref, wl_ref, wh_ref, b_ref, o_ref):
    acc = lax.dot_general(a0_ref[...], wl_ref[...],
                          (((1,), (1,)), ((), ())),
                          preferred_element_type=jnp.float32)
    acc += lax.dot_general(a1_ref[...], wh_ref[...],
                           (((1,), (1,)), ((), ())),
                           preferred_element_type=jnp.float32)
    o_ref[...] = acc + b_ref[...]


@jax.jit
def kernel(edge_index, h, W, b):
    src = edge_index[0]
    dst = edge_index[1]
    src_t = src.reshape(NS, NCHUNK, CHUNK)
    src2 = jnp.stack([2 * src_t, 2 * src_t + 1])
    dst_t = dst.reshape(NS, NCHUNK, CHUNK)
    h2 = h.reshape(2 * N, DH)
    zeros = jnp.zeros((ROW_LEN, DH), jnp.float32)

    agg2 = _sc_aggregate(src2, dst_t, h2, zeros)

    out = pl.pallas_call(
        _tc_matmul_body,
        grid=(N // _ROW_BLK,),
        in_specs=[
            pl.BlockSpec((_ROW_BLK, DH), lambda i: (i, 0)),
            pl.BlockSpec((_ROW_BLK, DH), lambda i: (i, 0)),
            pl.BlockSpec((H, DH), lambda i: (0, 0)),
            pl.BlockSpec((H, DH), lambda i: (0, 0)),
            pl.BlockSpec((1, H), lambda i: (0, 0)),
        ],
        out_specs=pl.BlockSpec((_ROW_BLK, H), lambda i: (i, 0)),
        out_shape=jax.ShapeDtypeStruct((N, H), jnp.float32),
    )(agg2[0], agg2[1], W[:, :DH], W[:, DH:], b.reshape(1, H))
    return out
